# baseline (speedup 1.0000x reference)
import functools

import jax
import jax.numpy as jnp
from jax import lax
from jax.experimental import pallas as pl
from jax.experimental.pallas import tpu as pltpu
from jax.experimental.pallas import tpu_sc as plsc

NC = 2
NS = 16
NW = NC * NS


B = 56
NR = 6
NI = 12
GA = 4
IA = 7
DD = NR - GA
F0 = 0.45


def _make_seg_sum(n, d, nfp0, nfp1):
    rps = (n // NS) // 8 * 8
    rtail = n - rps * NS
    assert rtail % 8 == 0
    epw0 = nfp0 * B
    epw1 = nfp1 * B
    assert B % 8 == 0 and NI % NR == 0
    for nfp in (nfp0, nfp1):
        assert nfp > NI + IA and (nfp - IA - NI) // NI >= 1

    mesh = plsc.VectorSubcoreMesh(core_axis_name="c", subcore_axis_name="s")

    @functools.partial(
        pl.kernel,
        mesh=mesh,
        out_type=jax.ShapeDtypeStruct((NC * n, d), jnp.float32),
        scratch_types=[
            [pltpu.VMEM((B,), jnp.int32) for _ in range(NI)],
            [pltpu.VMEM((B,), jnp.int32) for _ in range(NI)],
            [pltpu.VMEM((B, d), jnp.float32) for _ in range(NR)],
            [pltpu.SemaphoreType.DMA for _ in range(NI)],
            [pltpu.SemaphoreType.DMA for _ in range(NR)],
            [pltpu.SemaphoreType.DMA for _ in range(NR)],
            pltpu.VMEM_SHARED((n + 8, d), jnp.float32),
        ],
    )
    def seg_sum(x_hbm, src_hbm, dst_hbm, zero_hbm, out_hbm,
                sb, db, rows, isem, gsem, ssem, s_sh):
        cid = lax.axis_index("c")
        sid = lax.axis_index("s")

        pltpu.sync_copy(zero_hbm.at[pl.ds(sid * rps, rps)],
                        s_sh.at[pl.ds(sid * rps, rps)])
        if rtail:
            @pl.when(sid == 0)
            def _():
                pltpu.sync_copy(zero_hbm.at[pl.ds(NS * rps, rtail)],
                                s_sh.at[pl.ds(NS * rps, rtail)])
        plsc.subcore_barrier()

        def run_pipeline(nfp, ebase):
            steady_hi = NI + NI * ((nfp - IA - NI) // NI)

            def idx_issue(c, s):
                pltpu.async_copy(src_hbm.at[pl.ds(ebase + c * B, B)],
                                 sb[s], isem[s])
                pltpu.async_copy(dst_hbm.at[pl.ds(ebase + c * B, B)],
                                 db[s], isem[s])

            def idx_wait(c, s):
                pltpu.make_async_copy(src_hbm.at[pl.ds(ebase + c * B, B)],
                                      sb[s], isem[s]).wait()
                pltpu.make_async_copy(dst_hbm.at[pl.ds(ebase + c * B, B)],
                                      db[s], isem[s]).wait()

            def gather_issue(s, b):
                pltpu.async_copy(x_hbm.at[sb[s]], rows[b], gsem[b])

            def gather_wait(s, b):
                pltpu.make_async_copy(x_hbm.at[sb[s]], rows[b],
                                      gsem[b]).wait()

            def scatter_issue(s, b):
                pltpu.async_copy(rows[b], s_sh.at[db[s]], ssem[b], add=True)

            def scatter_wait(s, b):
                pltpu.make_async_copy(rows[b], s_sh.at[db[s]],
                                      ssem[b]).wait()

            def chunk_body(j, k8, drain=True, do_gather=True, do_idx=True):
                b = k8 % NR
                if drain:
                    sp = (k8 + NI - DD) % NI
                    scatter_wait(sp, sp % NR)
                if do_gather:
                    sg = (k8 + GA) % NI
                    idx_wait(j + GA, sg)
                    gather_issue(sg, sg % NR)
                if do_idx:
                    idx_issue(j + IA, (k8 + IA) % NI)
                gather_wait(k8, b)
                scatter_issue(k8, b)

            for c in range(IA):
                idx_issue(c, c)
            for c in range(GA):
                idx_wait(c, c)
                gather_issue(c, c % NR)
            for j in range(NI):
                chunk_body(j, j, drain=(j >= DD))

            @pl.loop(NI, steady_hi, step=NI)
            def _steady(jb):
                for k in range(NI):
                    chunk_body(jb + k, k)

            for j in range(steady_hi, nfp):
                chunk_body(j, j % NI,
                           do_gather=(j + GA < nfp), do_idx=(j + IA < nfp))
            for c in range(nfp - DD, nfp):
                scatter_wait(c % NI, c % NR)

        @pl.when(cid == 0)
        def _():
            run_pipeline(nfp0, sid * epw0)

        @pl.when(cid == 1)
        def _():
            run_pipeline(nfp1, NS * epw0 + sid * epw1)

        plsc.subcore_barrier()

        pltpu.sync_copy(s_sh.at[pl.ds(sid * rps, rps)],
                        out_hbm.at[pl.ds(cid * n + sid * rps, rps)])
        if rtail:
            @pl.when(sid == 0)
            def _():
                pltpu.sync_copy(s_sh.at[pl.ds(NS * rps, rtail)],
                                out_hbm.at[pl.ds(cid * n + NS * rps, rtail)])

    return seg_sum


def _mlp_body(x_ref, s2_ref, alpha_ref,
              w1_ref, b1_ref, g1_ref, be1_ref,
              w2_ref, b2_ref, g2_ref, be2_ref,
              w3_ref, b3_ref, o_ref):
    n = x_ref.shape[0]
    s = s2_ref[:n, :] + s2_ref[n:, :]
    z = alpha_ref[0, 0] * x_ref[...] + s

    def bn_relu(h, g, be):
        m = jnp.mean(h, axis=0, keepdims=True)
        dev = h - m
        v = jnp.mean(dev * dev, axis=0, keepdims=True)
        return jnp.maximum(g * (dev * lax.rsqrt(v + 1e-5)) + be, 0.0)

    h = jnp.dot(z, w1_ref[...], preferred_element_type=jnp.float32) + b1_ref[...]
    h = bn_relu(h, g1_ref[...], be1_ref[...])
    h = jnp.dot(h, w2_ref[...], preferred_element_type=jnp.float32) + b2_ref[...]
    h = bn_relu(h, g2_ref[...], be2_ref[...])
    o_ref[...] = (jnp.dot(h, w3_ref[...], preferred_element_type=jnp.float32)
                  + b3_ref[...])


def _make_mlp(n, d):
    smem_spec = pl.BlockSpec(memory_space=pltpu.SMEM)
    return pl.pallas_call(
        _mlp_body,
        out_shape=jax.ShapeDtypeStruct((n, d), jnp.float32),
        in_specs=[pl.BlockSpec(), pl.BlockSpec(), smem_spec]
                 + [pl.BlockSpec()] * 10,
    )


def kernel(X, edge_index, params):
    n, d = X.shape
    e = edge_index.shape[1]
    total = -(-e // (NS * B))
    nfp0 = int(total * F0)
    nfp1 = total - nfp0
    pad = NS * (nfp0 + nfp1) * B - e
    src = jnp.concatenate([edge_index[0], jnp.zeros((pad,), jnp.int32)])
    dst = jnp.concatenate([edge_index[1], jnp.full((pad,), n, jnp.int32)])
    zeros = jnp.zeros((n, d), jnp.float32)

    seg_sum = _make_seg_sum(n, d, nfp0, nfp1)
    mlp = _make_mlp(n, d)

    for p in params:
        s2 = seg_sum(X, src, dst, zeros)
        alpha = (1.0 + p['eps']).reshape(1, 1)
        X = mlp(X, s2, alpha,
                p['W1'].T, p['b1'].reshape(1, -1),
                p['g1'].reshape(1, -1), p['be1'].reshape(1, -1),
                p['W2'].T, p['b2'].reshape(1, -1),
                p['g2'].reshape(1, -1), p['be2'].reshape(1, -1),
                p['W3'].T, p['b3'].reshape(1, -1))
    return X


# speedup vs baseline: 1.0436x; 1.0436x over previous
import functools

import jax
import jax.numpy as jnp
from jax import lax
from jax.experimental import pallas as pl
from jax.experimental.pallas import tpu as pltpu
from jax.experimental.pallas import tpu_sc as plsc

NC = 2
NS = 16
NW = NC * NS


B = 56
NR = 6
NI = 12
GA = 4
IA = 7
DD = NR - GA
F0 = 0.55


def _make_seg_sum(n, d, nfp0, nfp1):
    rps = (n // NS) // 8 * 8
    rtail = n - rps * NS
    assert rtail % 8 == 0
    epw0 = nfp0 * B
    epw1 = nfp1 * B
    assert B % 8 == 0 and NI % NR == 0
    for nfp in (nfp0, nfp1):
        assert nfp > NI + IA and (nfp - IA - NI) // NI >= 1

    mesh = plsc.VectorSubcoreMesh(core_axis_name="c", subcore_axis_name="s")

    @functools.partial(
        pl.kernel,
        mesh=mesh,
        out_type=jax.ShapeDtypeStruct((NC * n, d), jnp.float32),
        scratch_types=[
            [pltpu.VMEM((B,), jnp.int32) for _ in range(NI)],
            [pltpu.VMEM((B,), jnp.int32) for _ in range(NI)],
            [pltpu.VMEM((B, d), jnp.float32) for _ in range(NR)],
            [pltpu.SemaphoreType.DMA for _ in range(NI)],
            [pltpu.SemaphoreType.DMA for _ in range(NR)],
            [pltpu.SemaphoreType.DMA for _ in range(NR)],
            pltpu.VMEM_SHARED((n + 8, d), jnp.float32),
        ],
    )
    def seg_sum(x_hbm, src_hbm, dst_hbm, zero_hbm, out_hbm,
                sb, db, rows, isem, gsem, ssem, s_sh):
        cid = lax.axis_index("c")
        sid = lax.axis_index("s")

        pltpu.sync_copy(zero_hbm.at[pl.ds(sid * rps, rps)],
                        s_sh.at[pl.ds(sid * rps, rps)])
        if rtail:
            @pl.when(sid == 0)
            def _():
                pltpu.sync_copy(zero_hbm.at[pl.ds(NS * rps, rtail)],
                                s_sh.at[pl.ds(NS * rps, rtail)])
        plsc.subcore_barrier()

        def run_pipeline(nfp, ebase):
            steady_hi = NI + NI * ((nfp - IA - NI) // NI)

            def idx_issue(c, s):
                pltpu.async_copy(src_hbm.at[pl.ds(ebase + c * B, B)],
                                 sb[s], isem[s])
                pltpu.async_copy(dst_hbm.at[pl.ds(ebase + c * B, B)],
                                 db[s], isem[s])

            def idx_wait(c, s):
                pltpu.make_async_copy(src_hbm.at[pl.ds(ebase + c * B, B)],
                                      sb[s], isem[s]).wait()
                pltpu.make_async_copy(dst_hbm.at[pl.ds(ebase + c * B, B)],
                                      db[s], isem[s]).wait()

            def gather_issue(s, b):
                pltpu.async_copy(x_hbm.at[sb[s]], rows[b], gsem[b])

            def gather_wait(s, b):
                pltpu.make_async_copy(x_hbm.at[sb[s]], rows[b],
                                      gsem[b]).wait()

            def scatter_issue(s, b):
                pltpu.async_copy(rows[b], s_sh.at[db[s]], ssem[b], add=True)

            def scatter_wait(s, b):
                pltpu.make_async_copy(rows[b], s_sh.at[db[s]],
                                      ssem[b]).wait()

            def chunk_body(j, k8, drain=True, do_gather=True, do_idx=True):
                b = k8 % NR
                if drain:
                    sp = (k8 + NI - DD) % NI
                    scatter_wait(sp, sp % NR)
                if do_gather:
                    sg = (k8 + GA) % NI
                    idx_wait(j + GA, sg)
                    gather_issue(sg, sg % NR)
                if do_idx:
                    idx_issue(j + IA, (k8 + IA) % NI)
                gather_wait(k8, b)
                scatter_issue(k8, b)

            for c in range(IA):
                idx_issue(c, c)
            for c in range(GA):
                idx_wait(c, c)
                gather_issue(c, c % NR)
            for j in range(NI):
                chunk_body(j, j, drain=(j >= DD))

            @pl.loop(NI, steady_hi, step=NI)
            def _steady(jb):
                for k in range(NI):
                    chunk_body(jb + k, k)

            for j in range(steady_hi, nfp):
                chunk_body(j, j % NI,
                           do_gather=(j + GA < nfp), do_idx=(j + IA < nfp))
            for c in range(nfp - DD, nfp):
                scatter_wait(c % NI, c % NR)

        @pl.when(cid == 0)
        def _():
            run_pipeline(nfp0, sid * epw0)

        @pl.when(cid == 1)
        def _():
            run_pipeline(nfp1, NS * epw0 + sid * epw1)

        plsc.subcore_barrier()

        pltpu.sync_copy(s_sh.at[pl.ds(sid * rps, rps)],
                        out_hbm.at[pl.ds(cid * n + sid * rps, rps)])
        if rtail:
            @pl.when(sid == 0)
            def _():
                pltpu.sync_copy(s_sh.at[pl.ds(NS * rps, rtail)],
                                out_hbm.at[pl.ds(cid * n + NS * rps, rtail)])

    return seg_sum


def _mlp_body(x_ref, s2_ref, alpha_ref,
              w1_ref, b1_ref, g1_ref, be1_ref,
              w2_ref, b2_ref, g2_ref, be2_ref,
              w3_ref, b3_ref, o_ref):
    n = x_ref.shape[0]
    s = s2_ref[:n, :] + s2_ref[n:, :]
    z = alpha_ref[0, 0] * x_ref[...] + s

    def bn_relu(h, g, be):
        m = jnp.mean(h, axis=0, keepdims=True)
        dev = h - m
        v = jnp.mean(dev * dev, axis=0, keepdims=True)
        return jnp.maximum(g * (dev * lax.rsqrt(v + 1e-5)) + be, 0.0)

    h = jnp.dot(z, w1_ref[...], preferred_element_type=jnp.float32) + b1_ref[...]
    h = bn_relu(h, g1_ref[...], be1_ref[...])
    h = jnp.dot(h, w2_ref[...], preferred_element_type=jnp.float32) + b2_ref[...]
    h = bn_relu(h, g2_ref[...], be2_ref[...])
    o_ref[...] = (jnp.dot(h, w3_ref[...], preferred_element_type=jnp.float32)
                  + b3_ref[...])


def _make_mlp(n, d):
    smem_spec = pl.BlockSpec(memory_space=pltpu.SMEM)
    return pl.pallas_call(
        _mlp_body,
        out_shape=jax.ShapeDtypeStruct((n, d), jnp.float32),
        in_specs=[pl.BlockSpec(), pl.BlockSpec(), smem_spec]
                 + [pl.BlockSpec()] * 10,
    )


def kernel(X, edge_index, params):
    n, d = X.shape
    e = edge_index.shape[1]
    total = -(-e // (NS * B))
    nfp0 = int(total * F0)
    nfp1 = total - nfp0
    pad = NS * (nfp0 + nfp1) * B - e
    src = jnp.concatenate([edge_index[0], jnp.zeros((pad,), jnp.int32)])
    dst = jnp.concatenate([edge_index[1], jnp.full((pad,), n, jnp.int32)])
    zeros = jnp.zeros((n, d), jnp.float32)

    seg_sum = _make_seg_sum(n, d, nfp0, nfp1)
    mlp = _make_mlp(n, d)

    for p in params:
        s2 = seg_sum(X, src, dst, zeros)
        alpha = (1.0 + p['eps']).reshape(1, 1)
        X = mlp(X, s2, alpha,
                p['W1'].T, p['b1'].reshape(1, -1),
                p['g1'].reshape(1, -1), p['be1'].reshape(1, -1),
                p['W2'].T, p['b2'].reshape(1, -1),
                p['g2'].reshape(1, -1), p['be2'].reshape(1, -1),
                p['W3'].T, p['b3'].reshape(1, -1))
    return X


# speedup vs baseline: 1.0840x; 1.0388x over previous
import functools

import jax
import jax.numpy as jnp
from jax import lax
from jax.experimental import pallas as pl
from jax.experimental.pallas import tpu as pltpu
from jax.experimental.pallas import tpu_sc as plsc

NC = 2
NS = 16
NW = NC * NS


B = 56
NR = 6
NI = 12
GA = 4
IA = 7
DD = NR - GA
F0 = 0.59


def _make_seg_sum(n, d, nfp0, nfp1):
    rps = (n // NS) // 8 * 8
    rtail = n - rps * NS
    assert rtail % 8 == 0
    epw0 = nfp0 * B
    epw1 = nfp1 * B
    assert B % 8 == 0 and NI % NR == 0
    for nfp in (nfp0, nfp1):
        assert nfp > NI + IA and (nfp - IA - NI) // NI >= 1

    mesh = plsc.VectorSubcoreMesh(core_axis_name="c", subcore_axis_name="s")

    @functools.partial(
        pl.kernel,
        mesh=mesh,
        out_type=jax.ShapeDtypeStruct((NC * n, d), jnp.float32),
        scratch_types=[
            [pltpu.VMEM((B,), jnp.int32) for _ in range(NI)],
            [pltpu.VMEM((B,), jnp.int32) for _ in range(NI)],
            [pltpu.VMEM((B, d), jnp.float32) for _ in range(NR)],
            [pltpu.SemaphoreType.DMA for _ in range(NI)],
            [pltpu.SemaphoreType.DMA for _ in range(NR)],
            [pltpu.SemaphoreType.DMA for _ in range(NR)],
            pltpu.VMEM_SHARED((n + 8, d), jnp.float32),
        ],
    )
    def seg_sum(x_hbm, src_hbm, dst_hbm, zero_hbm, out_hbm,
                sb, db, rows, isem, gsem, ssem, s_sh):
        cid = lax.axis_index("c")
        sid = lax.axis_index("s")

        pltpu.sync_copy(zero_hbm.at[pl.ds(sid * rps, rps)],
                        s_sh.at[pl.ds(sid * rps, rps)])
        if rtail:
            @pl.when(sid == 0)
            def _():
                pltpu.sync_copy(zero_hbm.at[pl.ds(NS * rps, rtail)],
                                s_sh.at[pl.ds(NS * rps, rtail)])
        plsc.subcore_barrier()

        def run_pipeline(nfp, ebase):
            steady_hi = NI + NI * ((nfp - IA - NI) // NI)

            def idx_issue(c, s):
                pltpu.async_copy(src_hbm.at[pl.ds(ebase + c * B, B)],
                                 sb[s], isem[s])
                pltpu.async_copy(dst_hbm.at[pl.ds(ebase + c * B, B)],
                                 db[s], isem[s])

            def idx_wait(c, s):
                pltpu.make_async_copy(src_hbm.at[pl.ds(ebase + c * B, B)],
                                      sb[s], isem[s]).wait()
                pltpu.make_async_copy(dst_hbm.at[pl.ds(ebase + c * B, B)],
                                      db[s], isem[s]).wait()

            def gather_issue(s, b):
                pltpu.async_copy(x_hbm.at[sb[s]], rows[b], gsem[b])

            def gather_wait(s, b):
                pltpu.make_async_copy(x_hbm.at[sb[s]], rows[b],
                                      gsem[b]).wait()

            def scatter_issue(s, b):
                pltpu.async_copy(rows[b], s_sh.at[db[s]], ssem[b], add=True)

            def scatter_wait(s, b):
                pltpu.make_async_copy(rows[b], s_sh.at[db[s]],
                                      ssem[b]).wait()

            def chunk_body(j, k8, drain=True, do_gather=True, do_idx=True):
                b = k8 % NR
                if drain:
                    sp = (k8 + NI - DD) % NI
                    scatter_wait(sp, sp % NR)
                if do_gather:
                    sg = (k8 + GA) % NI
                    idx_wait(j + GA, sg)
                    gather_issue(sg, sg % NR)
                if do_idx:
                    idx_issue(j + IA, (k8 + IA) % NI)
                gather_wait(k8, b)
                scatter_issue(k8, b)

            for c in range(IA):
                idx_issue(c, c)
            for c in range(GA):
                idx_wait(c, c)
                gather_issue(c, c % NR)
            for j in range(NI):
                chunk_body(j, j, drain=(j >= DD))

            @pl.loop(NI, steady_hi, step=NI)
            def _steady(jb):
                for k in range(NI):
                    chunk_body(jb + k, k)

            for j in range(steady_hi, nfp):
                chunk_body(j, j % NI,
                           do_gather=(j + GA < nfp), do_idx=(j + IA < nfp))
            for c in range(nfp - DD, nfp):
                scatter_wait(c % NI, c % NR)

        @pl.when(cid == 0)
        def _():
            run_pipeline(nfp0, sid * epw0)

        @pl.when(cid == 1)
        def _():
            run_pipeline(nfp1, NS * epw0 + sid * epw1)

        plsc.subcore_barrier()

        pltpu.sync_copy(s_sh.at[pl.ds(sid * rps, rps)],
                        out_hbm.at[pl.ds(cid * n + sid * rps, rps)])
        if rtail:
            @pl.when(sid == 0)
            def _():
                pltpu.sync_copy(s_sh.at[pl.ds(NS * rps, rtail)],
                                out_hbm.at[pl.ds(cid * n + NS * rps, rtail)])

    return seg_sum


def _mlp_body(x_ref, s2_ref, alpha_ref,
              w1_ref, b1_ref, g1_ref, be1_ref,
              w2_ref, b2_ref, g2_ref, be2_ref,
              w3_ref, b3_ref, o_ref):
    n = x_ref.shape[0]
    s = s2_ref[:n, :] + s2_ref[n:, :]
    z = alpha_ref[0, 0] * x_ref[...] + s

    def bn_relu(h, g, be):
        m = jnp.mean(h, axis=0, keepdims=True)
        dev = h - m
        v = jnp.mean(dev * dev, axis=0, keepdims=True)
        return jnp.maximum(g * (dev * lax.rsqrt(v + 1e-5)) + be, 0.0)

    h = jnp.dot(z, w1_ref[...], preferred_element_type=jnp.float32) + b1_ref[...]
    h = bn_relu(h, g1_ref[...], be1_ref[...])
    h = jnp.dot(h, w2_ref[...], preferred_element_type=jnp.float32) + b2_ref[...]
    h = bn_relu(h, g2_ref[...], be2_ref[...])
    o_ref[...] = (jnp.dot(h, w3_ref[...], preferred_element_type=jnp.float32)
                  + b3_ref[...])


def _make_mlp(n, d):
    smem_spec = pl.BlockSpec(memory_space=pltpu.SMEM)
    return pl.pallas_call(
        _mlp_body,
        out_shape=jax.ShapeDtypeStruct((n, d), jnp.float32),
        in_specs=[pl.BlockSpec(), pl.BlockSpec(), smem_spec]
                 + [pl.BlockSpec()] * 10,
    )


def kernel(X, edge_index, params):
    n, d = X.shape
    e = edge_index.shape[1]
    total = -(-e // (NS * B))
    nfp0 = int(total * F0)
    nfp1 = total - nfp0
    pad = NS * (nfp0 + nfp1) * B - e
    src = jnp.concatenate([edge_index[0], jnp.zeros((pad,), jnp.int32)])
    dst = jnp.concatenate([edge_index[1], jnp.full((pad,), n, jnp.int32)])
    zeros = jnp.zeros((n, d), jnp.float32)

    seg_sum = _make_seg_sum(n, d, nfp0, nfp1)
    mlp = _make_mlp(n, d)

    for p in params:
        s2 = seg_sum(X, src, dst, zeros)
        alpha = (1.0 + p['eps']).reshape(1, 1)
        X = mlp(X, s2, alpha,
                p['W1'].T, p['b1'].reshape(1, -1),
                p['g1'].reshape(1, -1), p['be1'].reshape(1, -1),
                p['W2'].T, p['b2'].reshape(1, -1),
                p['g2'].reshape(1, -1), p['be2'].reshape(1, -1),
                p['W3'].T, p['b3'].reshape(1, -1))
    return X


# speedup vs baseline: 1.1161x; 1.0296x over previous
import functools

import jax
import jax.numpy as jnp
from jax import lax
from jax.experimental import pallas as pl
from jax.experimental.pallas import tpu as pltpu
from jax.experimental.pallas import tpu_sc as plsc

NC = 2
NS = 16
NW = NC * NS


B = 56
NR = 6
NI = 12
GA = 4
IA = 7
DD = NR - GA
F0 = 0.63


def _make_seg_sum(n, d, nfp0, nfp1):
    rps = (n // NS) // 8 * 8
    rtail = n - rps * NS
    assert rtail % 8 == 0
    epw0 = nfp0 * B
    epw1 = nfp1 * B
    assert B % 8 == 0 and NI % NR == 0
    for nfp in (nfp0, nfp1):
        assert nfp > NI + IA and (nfp - IA - NI) // NI >= 1

    mesh = plsc.VectorSubcoreMesh(core_axis_name="c", subcore_axis_name="s")

    @functools.partial(
        pl.kernel,
        mesh=mesh,
        out_type=jax.ShapeDtypeStruct((NC * n, d), jnp.float32),
        scratch_types=[
            [pltpu.VMEM((B,), jnp.int32) for _ in range(NI)],
            [pltpu.VMEM((B,), jnp.int32) for _ in range(NI)],
            [pltpu.VMEM((B, d), jnp.float32) for _ in range(NR)],
            [pltpu.SemaphoreType.DMA for _ in range(NI)],
            [pltpu.SemaphoreType.DMA for _ in range(NR)],
            [pltpu.SemaphoreType.DMA for _ in range(NR)],
            pltpu.VMEM_SHARED((n + 8, d), jnp.float32),
        ],
    )
    def seg_sum(x_hbm, src_hbm, dst_hbm, zero_hbm, out_hbm,
                sb, db, rows, isem, gsem, ssem, s_sh):
        cid = lax.axis_index("c")
        sid = lax.axis_index("s")

        pltpu.sync_copy(zero_hbm.at[pl.ds(sid * rps, rps)],
                        s_sh.at[pl.ds(sid * rps, rps)])
        if rtail:
            @pl.when(sid == 0)
            def _():
                pltpu.sync_copy(zero_hbm.at[pl.ds(NS * rps, rtail)],
                                s_sh.at[pl.ds(NS * rps, rtail)])
        plsc.subcore_barrier()

        def run_pipeline(nfp, ebase):
            steady_hi = NI + NI * ((nfp - IA - NI) // NI)

            def idx_issue(c, s):
                pltpu.async_copy(src_hbm.at[pl.ds(ebase + c * B, B)],
                                 sb[s], isem[s])
                pltpu.async_copy(dst_hbm.at[pl.ds(ebase + c * B, B)],
                                 db[s], isem[s])

            def idx_wait(c, s):
                pltpu.make_async_copy(src_hbm.at[pl.ds(ebase + c * B, B)],
                                      sb[s], isem[s]).wait()
                pltpu.make_async_copy(dst_hbm.at[pl.ds(ebase + c * B, B)],
                                      db[s], isem[s]).wait()

            def gather_issue(s, b):
                pltpu.async_copy(x_hbm.at[sb[s]], rows[b], gsem[b])

            def gather_wait(s, b):
                pltpu.make_async_copy(x_hbm.at[sb[s]], rows[b],
                                      gsem[b]).wait()

            def scatter_issue(s, b):
                pltpu.async_copy(rows[b], s_sh.at[db[s]], ssem[b], add=True)

            def scatter_wait(s, b):
                pltpu.make_async_copy(rows[b], s_sh.at[db[s]],
                                      ssem[b]).wait()

            def chunk_body(j, k8, drain=True, do_gather=True, do_idx=True):
                b = k8 % NR
                if drain:
                    sp = (k8 + NI - DD) % NI
                    scatter_wait(sp, sp % NR)
                if do_gather:
                    sg = (k8 + GA) % NI
                    idx_wait(j + GA, sg)
                    gather_issue(sg, sg % NR)
                if do_idx:
                    idx_issue(j + IA, (k8 + IA) % NI)
                gather_wait(k8, b)
                scatter_issue(k8, b)

            for c in range(IA):
                idx_issue(c, c)
            for c in range(GA):
                idx_wait(c, c)
                gather_issue(c, c % NR)
            for j in range(NI):
                chunk_body(j, j, drain=(j >= DD))

            @pl.loop(NI, steady_hi, step=NI)
            def _steady(jb):
                for k in range(NI):
                    chunk_body(jb + k, k)

            for j in range(steady_hi, nfp):
                chunk_body(j, j % NI,
                           do_gather=(j + GA < nfp), do_idx=(j + IA < nfp))
            for c in range(nfp - DD, nfp):
                scatter_wait(c % NI, c % NR)

        @pl.when(cid == 0)
        def _():
            run_pipeline(nfp0, sid * epw0)

        @pl.when(cid == 1)
        def _():
            run_pipeline(nfp1, NS * epw0 + sid * epw1)

        plsc.subcore_barrier()

        pltpu.sync_copy(s_sh.at[pl.ds(sid * rps, rps)],
                        out_hbm.at[pl.ds(cid * n + sid * rps, rps)])
        if rtail:
            @pl.when(sid == 0)
            def _():
                pltpu.sync_copy(s_sh.at[pl.ds(NS * rps, rtail)],
                                out_hbm.at[pl.ds(cid * n + NS * rps, rtail)])

    return seg_sum


def _mlp_body(x_ref, s2_ref, alpha_ref,
              w1_ref, b1_ref, g1_ref, be1_ref,
              w2_ref, b2_ref, g2_ref, be2_ref,
              w3_ref, b3_ref, o_ref):
    n = x_ref.shape[0]
    s = s2_ref[:n, :] + s2_ref[n:, :]
    z = alpha_ref[0, 0] * x_ref[...] + s

    def bn_relu(h, g, be):
        m = jnp.mean(h, axis=0, keepdims=True)
        dev = h - m
        v = jnp.mean(dev * dev, axis=0, keepdims=True)
        return jnp.maximum(g * (dev * lax.rsqrt(v + 1e-5)) + be, 0.0)

    h = jnp.dot(z, w1_ref[...], preferred_element_type=jnp.float32) + b1_ref[...]
    h = bn_relu(h, g1_ref[...], be1_ref[...])
    h = jnp.dot(h, w2_ref[...], preferred_element_type=jnp.float32) + b2_ref[...]
    h = bn_relu(h, g2_ref[...], be2_ref[...])
    o_ref[...] = (jnp.dot(h, w3_ref[...], preferred_element_type=jnp.float32)
                  + b3_ref[...])


def _make_mlp(n, d):
    smem_spec = pl.BlockSpec(memory_space=pltpu.SMEM)
    return pl.pallas_call(
        _mlp_body,
        out_shape=jax.ShapeDtypeStruct((n, d), jnp.float32),
        in_specs=[pl.BlockSpec(), pl.BlockSpec(), smem_spec]
                 + [pl.BlockSpec()] * 10,
    )


def kernel(X, edge_index, params):
    n, d = X.shape
    e = edge_index.shape[1]
    total = -(-e // (NS * B))
    nfp0 = int(total * F0)
    nfp1 = total - nfp0
    pad = NS * (nfp0 + nfp1) * B - e
    src = jnp.concatenate([edge_index[0], jnp.zeros((pad,), jnp.int32)])
    dst = jnp.concatenate([edge_index[1], jnp.full((pad,), n, jnp.int32)])
    zeros = jnp.zeros((n, d), jnp.float32)

    seg_sum = _make_seg_sum(n, d, nfp0, nfp1)
    mlp = _make_mlp(n, d)

    for p in params:
        s2 = seg_sum(X, src, dst, zeros)
        alpha = (1.0 + p['eps']).reshape(1, 1)
        X = mlp(X, s2, alpha,
                p['W1'].T, p['b1'].reshape(1, -1),
                p['g1'].reshape(1, -1), p['be1'].reshape(1, -1),
                p['W2'].T, p['b2'].reshape(1, -1),
                p['g2'].reshape(1, -1), p['be2'].reshape(1, -1),
                p['W3'].T, p['b3'].reshape(1, -1))
    return X


# speedup vs baseline: 1.1573x; 1.0369x over previous
import functools

import jax
import jax.numpy as jnp
from jax import lax
from jax.experimental import pallas as pl
from jax.experimental.pallas import tpu as pltpu
from jax.experimental.pallas import tpu_sc as plsc

NC = 2
NS = 16
NW = NC * NS


B = 48
NR = 7
NI = 14
GA = 5
IA = 8
DD = NR - GA
F0 = 0.63


def _make_seg_sum(n, d, nfp0, nfp1):
    rps = (n // NS) // 8 * 8
    rtail = n - rps * NS
    assert rtail % 8 == 0
    epw0 = nfp0 * B
    epw1 = nfp1 * B
    assert B % 8 == 0 and NI % NR == 0
    for nfp in (nfp0, nfp1):
        assert nfp > NI + IA and (nfp - IA - NI) // NI >= 1

    mesh = plsc.VectorSubcoreMesh(core_axis_name="c", subcore_axis_name="s")

    @functools.partial(
        pl.kernel,
        mesh=mesh,
        out_type=jax.ShapeDtypeStruct((NC * n, d), jnp.float32),
        scratch_types=[
            [pltpu.VMEM((B,), jnp.int32) for _ in range(NI)],
            [pltpu.VMEM((B,), jnp.int32) for _ in range(NI)],
            [pltpu.VMEM((B, d), jnp.float32) for _ in range(NR)],
            [pltpu.SemaphoreType.DMA for _ in range(NI)],
            [pltpu.SemaphoreType.DMA for _ in range(NR)],
            [pltpu.SemaphoreType.DMA for _ in range(NR)],
            pltpu.VMEM_SHARED((n + 8, d), jnp.float32),
        ],
    )
    def seg_sum(x_hbm, src_hbm, dst_hbm, zero_hbm, out_hbm,
                sb, db, rows, isem, gsem, ssem, s_sh):
        cid = lax.axis_index("c")
        sid = lax.axis_index("s")

        pltpu.sync_copy(zero_hbm.at[pl.ds(sid * rps, rps)],
                        s_sh.at[pl.ds(sid * rps, rps)])
        if rtail:
            @pl.when(sid == 0)
            def _():
                pltpu.sync_copy(zero_hbm.at[pl.ds(NS * rps, rtail)],
                                s_sh.at[pl.ds(NS * rps, rtail)])
        plsc.subcore_barrier()

        def run_pipeline(nfp, ebase):
            steady_hi = NI + NI * ((nfp - IA - NI) // NI)

            def idx_issue(c, s):
                pltpu.async_copy(src_hbm.at[pl.ds(ebase + c * B, B)],
                                 sb[s], isem[s])
                pltpu.async_copy(dst_hbm.at[pl.ds(ebase + c * B, B)],
                                 db[s], isem[s])

            def idx_wait(c, s):
                pltpu.make_async_copy(src_hbm.at[pl.ds(ebase + c * B, B)],
                                      sb[s], isem[s]).wait()
                pltpu.make_async_copy(dst_hbm.at[pl.ds(ebase + c * B, B)],
                                      db[s], isem[s]).wait()

            def gather_issue(s, b):
                pltpu.async_copy(x_hbm.at[sb[s]], rows[b], gsem[b])

            def gather_wait(s, b):
                pltpu.make_async_copy(x_hbm.at[sb[s]], rows[b],
                                      gsem[b]).wait()

            def scatter_issue(s, b):
                pltpu.async_copy(rows[b], s_sh.at[db[s]], ssem[b], add=True)

            def scatter_wait(s, b):
                pltpu.make_async_copy(rows[b], s_sh.at[db[s]],
                                      ssem[b]).wait()

            def chunk_body(j, k8, drain=True, do_gather=True, do_idx=True):
                b = k8 % NR
                if drain:
                    sp = (k8 + NI - DD) % NI
                    scatter_wait(sp, sp % NR)
                if do_gather:
                    sg = (k8 + GA) % NI
                    idx_wait(j + GA, sg)
                    gather_issue(sg, sg % NR)
                if do_idx:
                    idx_issue(j + IA, (k8 + IA) % NI)
                gather_wait(k8, b)
                scatter_issue(k8, b)

            for c in range(IA):
                idx_issue(c, c)
            for c in range(GA):
                idx_wait(c, c)
                gather_issue(c, c % NR)
            for j in range(NI):
                chunk_body(j, j, drain=(j >= DD))

            @pl.loop(NI, steady_hi, step=NI)
            def _steady(jb):
                for k in range(NI):
                    chunk_body(jb + k, k)

            for j in range(steady_hi, nfp):
                chunk_body(j, j % NI,
                           do_gather=(j + GA < nfp), do_idx=(j + IA < nfp))
            for c in range(nfp - DD, nfp):
                scatter_wait(c % NI, c % NR)

        @pl.when(cid == 0)
        def _():
            run_pipeline(nfp0, sid * epw0)

        @pl.when(cid == 1)
        def _():
            run_pipeline(nfp1, NS * epw0 + sid * epw1)

        plsc.subcore_barrier()

        pltpu.sync_copy(s_sh.at[pl.ds(sid * rps, rps)],
                        out_hbm.at[pl.ds(cid * n + sid * rps, rps)])
        if rtail:
            @pl.when(sid == 0)
            def _():
                pltpu.sync_copy(s_sh.at[pl.ds(NS * rps, rtail)],
                                out_hbm.at[pl.ds(cid * n + NS * rps, rtail)])

    return seg_sum


def _mlp_body(x_ref, s2_ref, alpha_ref,
              w1_ref, b1_ref, g1_ref, be1_ref,
              w2_ref, b2_ref, g2_ref, be2_ref,
              w3_ref, b3_ref, o_ref):
    n = x_ref.shape[0]
    s = s2_ref[:n, :] + s2_ref[n:, :]
    z = alpha_ref[0, 0] * x_ref[...] + s

    def bn_relu(h, g, be):
        m = jnp.mean(h, axis=0, keepdims=True)
        dev = h - m
        v = jnp.mean(dev * dev, axis=0, keepdims=True)
        return jnp.maximum(g * (dev * lax.rsqrt(v + 1e-5)) + be, 0.0)

    h = jnp.dot(z, w1_ref[...], preferred_element_type=jnp.float32) + b1_ref[...]
    h = bn_relu(h, g1_ref[...], be1_ref[...])
    h = jnp.dot(h, w2_ref[...], preferred_element_type=jnp.float32) + b2_ref[...]
    h = bn_relu(h, g2_ref[...], be2_ref[...])
    o_ref[...] = (jnp.dot(h, w3_ref[...], preferred_element_type=jnp.float32)
                  + b3_ref[...])


def _make_mlp(n, d):
    smem_spec = pl.BlockSpec(memory_space=pltpu.SMEM)
    return pl.pallas_call(
        _mlp_body,
        out_shape=jax.ShapeDtypeStruct((n, d), jnp.float32),
        in_specs=[pl.BlockSpec(), pl.BlockSpec(), smem_spec]
                 + [pl.BlockSpec()] * 10,
    )


def kernel(X, edge_index, params):
    n, d = X.shape
    e = edge_index.shape[1]
    total = -(-e // (NS * B))
    nfp0 = int(total * F0)
    nfp1 = total - nfp0
    pad = NS * (nfp0 + nfp1) * B - e
    src = jnp.concatenate([edge_index[0], jnp.zeros((pad,), jnp.int32)])
    dst = jnp.concatenate([edge_index[1], jnp.full((pad,), n, jnp.int32)])
    zeros = jnp.zeros((n, d), jnp.float32)

    seg_sum = _make_seg_sum(n, d, nfp0, nfp1)
    mlp = _make_mlp(n, d)

    for p in params:
        s2 = seg_sum(X, src, dst, zeros)
        alpha = (1.0 + p['eps']).reshape(1, 1)
        X = mlp(X, s2, alpha,
                p['W1'].T, p['b1'].reshape(1, -1),
                p['g1'].reshape(1, -1), p['be1'].reshape(1, -1),
                p['W2'].T, p['b2'].reshape(1, -1),
                p['g2'].reshape(1, -1), p['be2'].reshape(1, -1),
                p['W3'].T, p['b3'].reshape(1, -1))
    return X
